# baseline (speedup 1.0000x reference)
import functools

import jax
import jax.numpy as jnp
from jax import lax
from jax.experimental import pallas as pl
from jax.experimental.pallas import tpu as pltpu
from jax.experimental.pallas import tpu_sc as plsc


_BL = 2048
_NBUF = 6


def _proj_body(vt_hbm, vtail_ref, w_ref, b_ref, out_hbm, vbuf, obuf, *sems):
    pre_dim, vocab = vt_hbm.shape
    nfull = vocab // _BL
    tail = vocab - nfull * _BL
    insems, outsems = sems[:_NBUF], sems[_NBUF:2 * _NBUF]
    w = w_ref[...]
    bvec = b_ref[...]
    dn = (((0,), (0,)), ((), ()))

    for b in range(_NBUF):
        pltpu.make_async_copy(
            vt_hbm.at[:, pl.ds(b * _BL, _BL)], vbuf.at[b], insems[b]).start()

    def body(k, carry):
        for b in range(_NBUF):
            step = k * _NBUF + b

            @pl.when(k >= 1)
            def _():
                pltpu.make_async_copy(
                    obuf.at[b], out_hbm.at[pl.ds(0, _BL)], outsems[b]).wait()

            pltpu.make_async_copy(
                vt_hbm.at[:, pl.ds(step * _BL, _BL)], vbuf.at[b],
                insems[b]).wait()
            obuf[b] = lax.dot_general(
                vbuf[b], w, dimension_numbers=dn,
                preferred_element_type=jnp.float32) + bvec
            pltpu.make_async_copy(
                obuf.at[b], out_hbm.at[pl.ds(step * _BL, _BL)],
                outsems[b]).start()
            nxt = step + _NBUF

            @pl.when(nxt < nfull)
            def _():
                pltpu.make_async_copy(
                    vt_hbm.at[:, pl.ds(nxt * _BL, _BL)], vbuf.at[b],
                    insems[b]).start()
        return carry

    lax.fori_loop(0, nfull // _NBUF, body, 0)
    for b in range(_NBUF):
        pltpu.make_async_copy(
            obuf.at[b], out_hbm.at[pl.ds(0, _BL)], outsems[b]).wait()
    if tail:
        obuf[0, pl.ds(0, tail)] = lax.dot_general(
            vtail_ref[...], w, dimension_numbers=dn,
            preferred_element_type=jnp.float32) + bvec
        pltpu.make_async_copy(
            obuf.at[0, pl.ds(0, tail)],
            out_hbm.at[pl.ds(nfull * _BL, tail)], outsems[0]).start()
        pltpu.make_async_copy(
            obuf.at[0, pl.ds(0, tail)],
            out_hbm.at[pl.ds(nfull * _BL, tail)], outsems[0]).wait()


def _project(vectors, W, bias):
    vocab, pre_dim = vectors.shape
    size = W.shape[0]
    nfull = vocab // _BL
    assert nfull % _NBUF == 0
    vt = vectors.T
    return pl.pallas_call(
        _proj_body,
        in_specs=[
            pl.BlockSpec(memory_space=pl.ANY),
            pl.BlockSpec(memory_space=pltpu.VMEM),
            pl.BlockSpec(memory_space=pltpu.VMEM),
            pl.BlockSpec(memory_space=pltpu.VMEM),
        ],
        out_specs=pl.BlockSpec(memory_space=pl.ANY),
        out_shape=jax.ShapeDtypeStruct((vocab, size), jnp.float32),
        scratch_shapes=(
            [pltpu.VMEM((_NBUF, pre_dim, _BL), jnp.float32),
             pltpu.VMEM((_NBUF, _BL, size), jnp.float32)]
            + [pltpu.SemaphoreType.DMA] * (2 * _NBUF)
        ),
    )(vt, vt[:, nfull * _BL:], W.T, bias.reshape(1, size))


_NC = 2
_NS = 16
_NW = _NC * _NS


@functools.lru_cache(maxsize=None)
def _make_gather(b_, l_, size):
    assert b_ % _NW == 0
    rpw = b_ // _NW
    mesh = plsc.VectorSubcoreMesh(core_axis_name="c", subcore_axis_name="s")

    nbuf = 4
    nouter = rpw // nbuf
    assert rpw % nbuf == 0

    @functools.partial(
        pl.kernel,
        mesh=mesh,
        out_type=jax.ShapeDtypeStruct((b_, l_, size), jnp.float32),
        scratch_types=[
            pltpu.VMEM((rpw, l_), jnp.int32),
            pltpu.VMEM((nbuf, l_, size), jnp.float32),
        ] + [pltpu.SemaphoreType.DMA] * nbuf,
    )
    def gather(tok_hbm, table_hbm, out_hbm, idx_v, rows_v, *sems):
        wid = lax.axis_index("s") * _NC + lax.axis_index("c")
        row0 = pl.multiple_of(wid * rpw, rpw)
        pltpu.sync_copy(tok_hbm.at[pl.ds(row0, rpw)], idx_v)

        def body(k, carry):
            for b in range(nbuf):
                c = k * nbuf + b

                @pl.when(k >= 1)
                def _():
                    pltpu.make_async_copy(
                        rows_v.at[b], out_hbm.at[row0], sems[b]).wait()

                pltpu.async_copy(
                    table_hbm.at[idx_v.at[c]], rows_v.at[b], sems[b])
            for b in range(nbuf):
                c = k * nbuf + b
                pltpu.make_async_copy(
                    table_hbm.at[idx_v.at[c]], rows_v.at[b], sems[b]).wait()
                pltpu.async_copy(
                    rows_v.at[b], out_hbm.at[row0 + c], sems[b])
            return carry

        lax.fori_loop(0, nouter, body, 0)
        for b in range(nbuf):
            pltpu.make_async_copy(
                rows_v.at[b], out_hbm.at[row0], sems[b]).wait()

    return gather


def kernel(tokens, vectors, W, bias):
    b_, l_ = tokens.shape
    size = W.shape[0]
    table = _project(vectors, W, bias)
    out = _make_gather(b_, l_, size)(tokens, table)
    return out


# speedup vs baseline: 1.5168x; 1.5168x over previous
import functools

import jax
import jax.numpy as jnp
from jax import lax
from jax.experimental import pallas as pl
from jax.experimental.pallas import tpu as pltpu
from jax.experimental.pallas import tpu_sc as plsc


_BL = 2048
_NBUF = 6


def _proj_body(vt_hbm, vtail_ref, w_ref, b_ref, out_hbm, vbuf, obuf, *sems):
    pre_dim, vocab = vt_hbm.shape
    nfull = vocab // _BL
    tail = vocab - nfull * _BL
    insems, outsems = sems[:_NBUF], sems[_NBUF:2 * _NBUF]
    w = w_ref[...]
    bvec = b_ref[...]
    dn = (((0,), (0,)), ((), ()))

    for b in range(_NBUF):
        pltpu.make_async_copy(
            vt_hbm.at[:, pl.ds(b * _BL, _BL)], vbuf.at[b], insems[b]).start()

    def body(k, carry):
        for b in range(_NBUF):
            step = k * _NBUF + b

            @pl.when(k >= 1)
            def _():
                pltpu.make_async_copy(
                    obuf.at[b], out_hbm.at[pl.ds(0, _BL)], outsems[b]).wait()

            pltpu.make_async_copy(
                vt_hbm.at[:, pl.ds(step * _BL, _BL)], vbuf.at[b],
                insems[b]).wait()
            obuf[b] = lax.dot_general(
                vbuf[b], w, dimension_numbers=dn,
                preferred_element_type=jnp.float32) + bvec
            pltpu.make_async_copy(
                obuf.at[b], out_hbm.at[pl.ds(step * _BL, _BL)],
                outsems[b]).start()
            nxt = step + _NBUF

            @pl.when(nxt < nfull)
            def _():
                pltpu.make_async_copy(
                    vt_hbm.at[:, pl.ds(nxt * _BL, _BL)], vbuf.at[b],
                    insems[b]).start()
        return carry

    lax.fori_loop(0, nfull // _NBUF, body, 0)
    for b in range(_NBUF):
        pltpu.make_async_copy(
            obuf.at[b], out_hbm.at[pl.ds(0, _BL)], outsems[b]).wait()
    if tail:
        obuf[0, pl.ds(0, tail)] = lax.dot_general(
            vtail_ref[...], w, dimension_numbers=dn,
            preferred_element_type=jnp.float32) + bvec
        pltpu.make_async_copy(
            obuf.at[0, pl.ds(0, tail)],
            out_hbm.at[pl.ds(nfull * _BL, tail)], outsems[0]).start()
        pltpu.make_async_copy(
            obuf.at[0, pl.ds(0, tail)],
            out_hbm.at[pl.ds(nfull * _BL, tail)], outsems[0]).wait()


def _project(vectors, W, bias):
    vocab, pre_dim = vectors.shape
    size = W.shape[0]
    nfull = vocab // _BL
    assert nfull % _NBUF == 0
    vt = vectors.T
    return pl.pallas_call(
        _proj_body,
        in_specs=[
            pl.BlockSpec(memory_space=pl.ANY),
            pl.BlockSpec(memory_space=pltpu.VMEM),
            pl.BlockSpec(memory_space=pltpu.VMEM),
            pl.BlockSpec(memory_space=pltpu.VMEM),
        ],
        out_specs=pl.BlockSpec(memory_space=pl.ANY),
        out_shape=jax.ShapeDtypeStruct((vocab, size), jnp.float32),
        scratch_shapes=(
            [pltpu.VMEM((_NBUF, pre_dim, _BL), jnp.float32),
             pltpu.VMEM((_NBUF, _BL, size), jnp.float32)]
            + [pltpu.SemaphoreType.DMA] * (2 * _NBUF)
        ),
    )(vt, vt[:, nfull * _BL:], W.T, bias.reshape(1, size))


_NC = 2
_NS = 16
_NW = _NC * _NS


@functools.lru_cache(maxsize=None)
def _make_gather(b_, l_, size):
    assert b_ % _NW == 0
    cpw = b_ // _NW
    mesh = plsc.VectorSubcoreMesh(core_axis_name="c", subcore_axis_name="s")

    nbuf = 5
    nouter = l_ // nbuf
    assert l_ % nbuf == 0

    @functools.partial(
        pl.kernel,
        mesh=mesh,
        out_type=jax.ShapeDtypeStruct((l_, b_, size), jnp.float32),
        scratch_types=[
            pltpu.VMEM((l_, cpw), jnp.int32),
            pltpu.VMEM((nbuf, cpw, size), jnp.float32),
        ] + [pltpu.SemaphoreType.DMA] * nbuf,
    )
    def gather(tokt_hbm, table_hbm, out_hbm, idx_v, rows_v, *sems):
        wid = lax.axis_index("s") * _NC + lax.axis_index("c")
        col0 = pl.multiple_of(wid * cpw, cpw)
        pltpu.sync_copy(tokt_hbm.at[:, pl.ds(col0, cpw)], idx_v)

        def body(k, carry):
            for b in range(nbuf):
                li = k * nbuf + b

                @pl.when(k >= 1)
                def _():
                    pltpu.make_async_copy(
                        rows_v.at[b], out_hbm.at[0, pl.ds(col0, cpw)],
                        sems[b]).wait()

                pltpu.async_copy(
                    table_hbm.at[idx_v.at[li]], rows_v.at[b], sems[b])
            for b in range(nbuf):
                li = k * nbuf + b
                pltpu.make_async_copy(
                    table_hbm.at[idx_v.at[li]], rows_v.at[b], sems[b]).wait()
                pltpu.async_copy(
                    rows_v.at[b], out_hbm.at[li, pl.ds(col0, cpw)], sems[b])
            return carry

        lax.fori_loop(0, nouter, body, 0)
        for b in range(nbuf):
            pltpu.make_async_copy(
                rows_v.at[b], out_hbm.at[0, pl.ds(col0, cpw)], sems[b]).wait()

    return gather


def kernel(tokens, vectors, W, bias):
    b_, l_ = tokens.shape
    size = W.shape[0]
    table = _project(vectors, W, bias)
    out = _make_gather(b_, l_, size)(tokens.T, table)
    return out.transpose(1, 0, 2)
